# Initial kernel scaffold; baseline (speedup 1.0000x reference)
import jax
import jax.numpy as jnp
from jax.experimental import pallas as pl


def kernel(hidden_states, ln_attn_w, Wq, Wk, Wv, Wo, ln_ff_w, Wi, Wo_ff, router_W):
    raise NotImplementedError("write your pallas kernel here")


# speedup vs baseline: 1.3083x; 1.3083x over previous
import functools

import jax
import jax.numpy as jnp
from jax import lax
from jax.experimental import pallas as pl
from jax.experimental.pallas import tpu as pltpu
from jax.experimental.pallas import tpu_sc as plsc

S, D = 2048, 768
H, DK = 12, 64
INNER = H * DK
DFF = 3072
E, TOPK = 8, 2
EPS = 1e-6

BQ = 256
TB = 128
P = 5120
NB = P // TB


def _dot(a, b):
    return jnp.dot(a, b, preferred_element_type=jnp.float32)


_LOG2E = 1.4426950408889634


def _exp(x):
    return jnp.exp2(x * _LOG2E)


def _row_mean_sq(x, d):
    s = x * x
    acc = s[:, 0:128]
    for c in range(128, d, 128):
        acc = acc + s[:, c:c + 128]
    w = 64
    while w >= 1:
        acc = acc[:, :w] + acc[:, w:2 * w]
        w //= 2
    return acc / d


def _ln_qkv_body(x_ref, lnw_ref, w_ref, out_ref):
    x = x_ref[...]
    var = _row_mean_sq(x, D)
    normed = x * lax.rsqrt(var + EPS) * lnw_ref[...]
    out_ref[...] = _dot(normed, w_ref[...])


def _attn_body(q_ref, k_ref, v_ref, o_ref):
    outs = []
    for h in range(2):
        q = q_ref[:, h * DK:(h + 1) * DK]
        k = k_ref[:, h * DK:(h + 1) * DK]
        s = lax.dot_general(q, k, (((1,), (1,)), ((), ())),
                            preferred_element_type=jnp.float32)
        m = jnp.max(s, axis=1, keepdims=True)
        p = _exp(s - m)
        a = p / jnp.sum(p, axis=1, keepdims=True)
        outs.append(_dot(a, v_ref[:, h * DK:(h + 1) * DK]))
    o_ref[...] = jnp.concatenate(outs, axis=1)


def _proj_router_body(ctx_ref, x_ref, wo_ref, rw_ref,
                      xt_ref, lg_ref, w2_ref, i2_ref):
    xt = x_ref[...] + _dot(ctx_ref[...], wo_ref[...])
    xt_ref[...] = xt
    lg = _dot(xt, rw_ref[...])
    lg_ref[...] = lg
    mx = jnp.max(lg, axis=1, keepdims=True)
    ex = _exp(lg - mx)
    p = ex / jnp.sum(ex, axis=1, keepdims=True)
    iota = lax.broadcasted_iota(jnp.int32, p.shape, 1)
    m1 = jnp.max(p, axis=1, keepdims=True)
    i1 = jnp.min(jnp.where(p >= m1, iota, E), axis=1, keepdims=True)
    pm = jnp.where(iota == i1, -1.0, p)
    m2 = jnp.max(pm, axis=1, keepdims=True)
    i2 = jnp.min(jnp.where(pm >= m2, iota, E), axis=1, keepdims=True)
    tot = m1 + m2
    w2_ref[...] = jnp.concatenate([m1 / tot, m2 / tot], axis=1)
    i2_ref[...] = jnp.concatenate([i1, i2], axis=1)


def _ffn_body(eid_ref, x_ref, lnw_ref, wi_ref, wo_ref, wr_ref, out_ref):
    x = x_ref[...]
    var = jnp.mean(x * x, axis=1, keepdims=True)
    h = x * lax.rsqrt(var + EPS) * lnw_ref[0]
    a = jnp.maximum(_dot(h, wi_ref[0]), 0.0)
    c = _dot(a, wo_ref[0])
    out_ref[...] = c * wr_ref[:, :1]


def _combine_body(xt_ref, g0_ref, g1_ref, o_ref):
    o_ref[...] = xt_ref[...] + g0_ref[...] + g1_ref[...]


_SC_CORES = 2
_SC_SUBCORES = 16


def _make_sc_gather(n_rows, d, b_idx, chunk):
    nw = _SC_CORES * _SC_SUBCORES
    b_per_w = b_idx // nw
    nch = b_per_w // chunk
    assert b_per_w % chunk == 0 and chunk <= 128 and chunk % 8 == 0
    mesh = plsc.VectorSubcoreMesh(core_axis_name="c", subcore_axis_name="s")

    @functools.partial(
        pl.kernel,
        out_type=jax.ShapeDtypeStruct((b_idx, d), jnp.float32),
        mesh=mesh,
        scratch_types=[
            pltpu.VMEM((chunk,), jnp.int32),
            pltpu.VMEM((chunk, d), jnp.float32),
            pltpu.SemaphoreType.DMA,
        ],
    )
    def gather(table_hbm, idx_hbm, out_hbm, idx_v, rows_v, sem):
        wid = lax.axis_index("s") * _SC_CORES + lax.axis_index("c")
        base = wid * b_per_w
        for c in range(nch):
            off = base + c * chunk
            pltpu.sync_copy(idx_hbm.at[pl.ds(off, chunk)], idx_v)
            pltpu.async_copy(table_hbm.at[idx_v], rows_v, sem).wait()
            pltpu.sync_copy(rows_v, out_hbm.at[pl.ds(off, chunk)])

    return gather


_gather_dispatch = _make_sc_gather(S, D, P, 80)
_gather_return = _make_sc_gather(P, D, 2 * S, 128)


def kernel(hidden_states, ln_attn_w, Wq, Wk, Wv, Wo, ln_ff_w, Wi, Wo_ff,
           router_W):
    x = hidden_states.reshape(S, D)
    wqkv = jnp.concatenate([Wq, Wk, Wv], axis=1)

    qkv = pl.pallas_call(
        _ln_qkv_body,
        grid=(S // BQ,),
        in_specs=[
            pl.BlockSpec((BQ, D), lambda i: (i, 0)),
            pl.BlockSpec((1, D), lambda i: (0, 0)),
            pl.BlockSpec((D, 3 * INNER), lambda i: (0, 0)),
        ],
        out_specs=pl.BlockSpec((BQ, 3 * INNER), lambda i: (i, 0)),
        out_shape=jax.ShapeDtypeStruct((S, 3 * INNER), jnp.float32),
    )(x, ln_attn_w.reshape(1, D), wqkv)

    HP = H // 2
    ctx = pl.pallas_call(
        _attn_body,
        grid=(HP, S // BQ),
        in_specs=[
            pl.BlockSpec((BQ, 2 * DK), lambda h, i: (i, h)),
            pl.BlockSpec((S, 2 * DK), lambda h, i: (0, HP + h)),
            pl.BlockSpec((S, 2 * DK), lambda h, i: (0, 2 * HP + h)),
        ],
        out_specs=pl.BlockSpec((BQ, 2 * DK), lambda h, i: (i, h)),
        out_shape=jax.ShapeDtypeStruct((S, INNER), jnp.float32),
    )(qkv, qkv, qkv)

    xt, router_logits, w2, i2 = pl.pallas_call(
        _proj_router_body,
        grid=(S // BQ,),
        in_specs=[
            pl.BlockSpec((BQ, INNER), lambda i: (i, 0)),
            pl.BlockSpec((BQ, D), lambda i: (i, 0)),
            pl.BlockSpec((INNER, D), lambda i: (0, 0)),
            pl.BlockSpec((D, E), lambda i: (0, 0)),
        ],
        out_specs=[
            pl.BlockSpec((BQ, D), lambda i: (i, 0)),
            pl.BlockSpec((BQ, E), lambda i: (i, 0)),
            pl.BlockSpec((BQ, TOPK), lambda i: (i, 0)),
            pl.BlockSpec((BQ, TOPK), lambda i: (i, 0)),
        ],
        out_shape=[
            jax.ShapeDtypeStruct((S, D), jnp.float32),
            jax.ShapeDtypeStruct((S, E), jnp.float32),
            jax.ShapeDtypeStruct((S, TOPK), jnp.float32),
            jax.ShapeDtypeStruct((S, TOPK), jnp.int32),
        ],
    )(ctx, x, Wo, router_W)

    e_flat = i2.reshape(-1)
    w_flat = w2.reshape(-1)
    onehot = (e_flat[:, None] == jnp.arange(E)).astype(jnp.int32)
    cnt = onehot.sum(0)
    rank = jnp.cumsum(onehot, axis=0) - onehot
    rank_i = jnp.take_along_axis(rank, e_flat[:, None], axis=1)[:, 0]
    pcnt = ((cnt + TB - 1) // TB) * TB
    poff = jnp.concatenate([jnp.zeros((1,), jnp.int32),
                            jnp.cumsum(pcnt)[:-1].astype(jnp.int32)])
    pos = poff[e_flat] + rank_i
    srctok = jnp.zeros((P,), jnp.int32).at[pos].set(
        jnp.arange(2 * S, dtype=jnp.int32) // 2)
    w_sorted = jnp.zeros((P,), jnp.float32).at[pos].set(w_flat)
    pend = (poff + pcnt).astype(jnp.int32)
    blk_eid = jnp.clip(
        jnp.sum((jnp.arange(NB, dtype=jnp.int32)[:, None] * TB) >= pend[None, :],
                axis=1), 0, E - 1).astype(jnp.int32)
    wrow = jnp.broadcast_to(w_sorted[:, None], (P, 128))
    pos01 = jnp.concatenate([pos[0::2], pos[1::2]])

    sorted_x = _gather_dispatch(xt, srctok)

    contrib = pl.pallas_call(
        _ffn_body,
        grid_spec=pltpu.PrefetchScalarGridSpec(
            num_scalar_prefetch=1,
            grid=(NB,),
            in_specs=[
                pl.BlockSpec((TB, D), lambda b, eid: (b, 0)),
                pl.BlockSpec((1, 1, D), lambda b, eid: (eid[b], 0, 0)),
                pl.BlockSpec((1, D, DFF), lambda b, eid: (eid[b], 0, 0)),
                pl.BlockSpec((1, DFF, D), lambda b, eid: (eid[b], 0, 0)),
                pl.BlockSpec((TB, 128), lambda b, eid: (b, 0)),
            ],
            out_specs=pl.BlockSpec((TB, D), lambda b, eid: (b, 0)),
        ),
        out_shape=jax.ShapeDtypeStruct((P, D), jnp.float32),
    )(blk_eid, sorted_x, ln_ff_w.reshape(E, 1, D), Wi, Wo_ff, wrow)

    g01 = _gather_return(contrib, pos01)

    out = pl.pallas_call(
        _combine_body,
        grid=(S // BQ,),
        in_specs=[
            pl.BlockSpec((BQ, D), lambda i: (i, 0)),
            pl.BlockSpec((BQ, D), lambda i: (i, 0)),
            pl.BlockSpec((BQ, D), lambda i: (i + S // BQ, 0)),
        ],
        out_specs=pl.BlockSpec((BQ, D), lambda i: (i, 0)),
        out_shape=jax.ShapeDtypeStruct((S, D), jnp.float32),
    )(xt, g01, g01)

    return out.reshape(1, S, D), router_logits


# speedup vs baseline: 1.3085x; 1.0001x over previous
import functools

import jax
import jax.numpy as jnp
from jax import lax
from jax.experimental import pallas as pl
from jax.experimental.pallas import tpu as pltpu
from jax.experimental.pallas import tpu_sc as plsc

S, D = 2048, 768
H, DK = 12, 64
INNER = H * DK
DFF = 3072
E, TOPK = 8, 2
EPS = 1e-6

BQ = 256
TB = 128
P = 5120
NB = P // TB


def _dot(a, b):
    return jnp.dot(a, b, preferred_element_type=jnp.float32)


_LOG2E = 1.4426950408889634


def _exp(x):
    return jnp.exp2(x * _LOG2E)


def _row_mean_sq(x, d):
    s = x * x
    acc = s[:, 0:128]
    for c in range(128, d, 128):
        acc = acc + s[:, c:c + 128]
    w = 64
    while w >= 1:
        acc = acc[:, :w] + acc[:, w:2 * w]
        w //= 2
    return acc / d


def _ln_qkv_body(x_ref, lnw_ref, w_ref, out_ref):
    x = x_ref[...]
    var = _row_mean_sq(x, D)
    normed = x * lax.rsqrt(var + EPS) * lnw_ref[...]
    out_ref[...] = _dot(normed, w_ref[...])


def _attn_body(q_ref, k_ref, v_ref, o_ref):
    outs = []
    for h in range(2):
        q = q_ref[:, h * DK:(h + 1) * DK]
        k = k_ref[:, h * DK:(h + 1) * DK]
        s = lax.dot_general(q, k, (((1,), (1,)), ((), ())),
                            preferred_element_type=jnp.float32)
        m = jnp.max(s, axis=1, keepdims=True)
        p = _exp(s - m)
        a = p / jnp.sum(p, axis=1, keepdims=True)
        outs.append(_dot(a, v_ref[:, h * DK:(h + 1) * DK]))
    o_ref[...] = jnp.concatenate(outs, axis=1)


def _proj_router_body(ctx_ref, x_ref, wo_ref, rw_ref,
                      xt_ref, lg_ref, w2_ref, i2_ref):
    xt = x_ref[...] + _dot(ctx_ref[...], wo_ref[...])
    xt_ref[...] = xt
    lg = _dot(xt, rw_ref[...])
    lg_ref[...] = lg
    mx = jnp.max(lg, axis=1, keepdims=True)
    ex = _exp(lg - mx)
    p = ex / jnp.sum(ex, axis=1, keepdims=True)
    iota = lax.broadcasted_iota(jnp.int32, p.shape, 1)
    m1 = jnp.max(p, axis=1, keepdims=True)
    i1 = jnp.min(jnp.where(p >= m1, iota, E), axis=1, keepdims=True)
    pm = jnp.where(iota == i1, -1.0, p)
    m2 = jnp.max(pm, axis=1, keepdims=True)
    i2 = jnp.min(jnp.where(pm >= m2, iota, E), axis=1, keepdims=True)
    tot = m1 + m2
    w2_ref[...] = jnp.concatenate([m1 / tot, m2 / tot], axis=1)
    i2_ref[...] = jnp.concatenate([i1, i2], axis=1)


def _ffn_body(eid_ref, x_ref, lnw_ref, wi_ref, wo_ref, wr_ref, out_ref):
    x = x_ref[...]
    var = jnp.mean(x * x, axis=1, keepdims=True)
    h = x * lax.rsqrt(var + EPS) * lnw_ref[0]
    a = jnp.maximum(_dot(h, wi_ref[0]), 0.0)
    c = _dot(a, wo_ref[0])
    out_ref[...] = c * wr_ref[:, :1]


def _combine_body(xt_ref, g0_ref, g1_ref, o_ref):
    o_ref[...] = xt_ref[...] + g0_ref[...] + g1_ref[...]


_SC_CORES = 2
_SC_SUBCORES = 16


def _make_sc_gather(n_rows, d, b_idx, chunk):
    nw = _SC_CORES * _SC_SUBCORES
    b_per_w = b_idx // nw
    nch = b_per_w // chunk
    assert b_per_w % chunk == 0 and chunk <= 160 and chunk % 8 == 0
    mesh = plsc.VectorSubcoreMesh(core_axis_name="c", subcore_axis_name="s")

    @functools.partial(
        pl.kernel,
        out_type=jax.ShapeDtypeStruct((b_idx, d), jnp.float32),
        mesh=mesh,
        scratch_types=[
            pltpu.VMEM((chunk,), jnp.int32),
            pltpu.VMEM((chunk, d), jnp.float32),
            pltpu.SemaphoreType.DMA,
        ],
    )
    def gather(table_hbm, idx_hbm, out_hbm, idx_v, rows_v, sem):
        wid = lax.axis_index("s") * _SC_CORES + lax.axis_index("c")
        base = wid * b_per_w
        for c in range(nch):
            off = base + c * chunk
            pltpu.sync_copy(idx_hbm.at[pl.ds(off, chunk)], idx_v)
            pltpu.async_copy(table_hbm.at[idx_v], rows_v, sem).wait()
            pltpu.sync_copy(rows_v, out_hbm.at[pl.ds(off, chunk)])

    return gather


_gather_dispatch = _make_sc_gather(S, D, P, 160)
_gather_return = _make_sc_gather(P, D, 2 * S, 128)


def kernel(hidden_states, ln_attn_w, Wq, Wk, Wv, Wo, ln_ff_w, Wi, Wo_ff,
           router_W):
    x = hidden_states.reshape(S, D)
    wqkv = jnp.concatenate([Wq, Wk, Wv], axis=1)

    qkv = pl.pallas_call(
        _ln_qkv_body,
        grid=(S // BQ,),
        in_specs=[
            pl.BlockSpec((BQ, D), lambda i: (i, 0)),
            pl.BlockSpec((1, D), lambda i: (0, 0)),
            pl.BlockSpec((D, 3 * INNER), lambda i: (0, 0)),
        ],
        out_specs=pl.BlockSpec((BQ, 3 * INNER), lambda i: (i, 0)),
        out_shape=jax.ShapeDtypeStruct((S, 3 * INNER), jnp.float32),
    )(x, ln_attn_w.reshape(1, D), wqkv)

    HP = H // 2
    ctx = pl.pallas_call(
        _attn_body,
        grid=(HP, S // BQ),
        in_specs=[
            pl.BlockSpec((BQ, 2 * DK), lambda h, i: (i, h)),
            pl.BlockSpec((S, 2 * DK), lambda h, i: (0, HP + h)),
            pl.BlockSpec((S, 2 * DK), lambda h, i: (0, 2 * HP + h)),
        ],
        out_specs=pl.BlockSpec((BQ, 2 * DK), lambda h, i: (i, h)),
        out_shape=jax.ShapeDtypeStruct((S, INNER), jnp.float32),
    )(qkv, qkv, qkv)

    xt, router_logits, w2, i2 = pl.pallas_call(
        _proj_router_body,
        grid=(S // BQ,),
        in_specs=[
            pl.BlockSpec((BQ, INNER), lambda i: (i, 0)),
            pl.BlockSpec((BQ, D), lambda i: (i, 0)),
            pl.BlockSpec((INNER, D), lambda i: (0, 0)),
            pl.BlockSpec((D, E), lambda i: (0, 0)),
        ],
        out_specs=[
            pl.BlockSpec((BQ, D), lambda i: (i, 0)),
            pl.BlockSpec((BQ, E), lambda i: (i, 0)),
            pl.BlockSpec((BQ, TOPK), lambda i: (i, 0)),
            pl.BlockSpec((BQ, TOPK), lambda i: (i, 0)),
        ],
        out_shape=[
            jax.ShapeDtypeStruct((S, D), jnp.float32),
            jax.ShapeDtypeStruct((S, E), jnp.float32),
            jax.ShapeDtypeStruct((S, TOPK), jnp.float32),
            jax.ShapeDtypeStruct((S, TOPK), jnp.int32),
        ],
    )(ctx, x, Wo, router_W)

    e_flat = i2.reshape(-1)
    w_flat = w2.reshape(-1)
    onehot = (e_flat[:, None] == jnp.arange(E)).astype(jnp.int32)
    cnt = onehot.sum(0)
    rank = jnp.cumsum(onehot, axis=0) - onehot
    rank_i = jnp.take_along_axis(rank, e_flat[:, None], axis=1)[:, 0]
    pcnt = ((cnt + TB - 1) // TB) * TB
    poff = jnp.concatenate([jnp.zeros((1,), jnp.int32),
                            jnp.cumsum(pcnt)[:-1].astype(jnp.int32)])
    pos = poff[e_flat] + rank_i
    srctok = jnp.zeros((P,), jnp.int32).at[pos].set(
        jnp.arange(2 * S, dtype=jnp.int32) // 2)
    w_sorted = jnp.zeros((P,), jnp.float32).at[pos].set(w_flat)
    pend = (poff + pcnt).astype(jnp.int32)
    blk_eid = jnp.clip(
        jnp.sum((jnp.arange(NB, dtype=jnp.int32)[:, None] * TB) >= pend[None, :],
                axis=1), 0, E - 1).astype(jnp.int32)
    wrow = jnp.broadcast_to(w_sorted[:, None], (P, 128))
    pos01 = jnp.concatenate([pos[0::2], pos[1::2]])

    sorted_x = _gather_dispatch(xt, srctok)

    contrib = pl.pallas_call(
        _ffn_body,
        grid_spec=pltpu.PrefetchScalarGridSpec(
            num_scalar_prefetch=1,
            grid=(NB,),
            in_specs=[
                pl.BlockSpec((TB, D), lambda b, eid: (b, 0)),
                pl.BlockSpec((1, 1, D), lambda b, eid: (eid[b], 0, 0)),
                pl.BlockSpec((1, D, DFF), lambda b, eid: (eid[b], 0, 0)),
                pl.BlockSpec((1, DFF, D), lambda b, eid: (eid[b], 0, 0)),
                pl.BlockSpec((TB, 128), lambda b, eid: (b, 0)),
            ],
            out_specs=pl.BlockSpec((TB, D), lambda b, eid: (b, 0)),
        ),
        out_shape=jax.ShapeDtypeStruct((P, D), jnp.float32),
    )(blk_eid, sorted_x, ln_ff_w.reshape(E, 1, D), Wi, Wo_ff, wrow)

    g01 = _gather_return(contrib, pos01)

    out = pl.pallas_call(
        _combine_body,
        grid=(S // BQ,),
        in_specs=[
            pl.BlockSpec((BQ, D), lambda i: (i, 0)),
            pl.BlockSpec((BQ, D), lambda i: (i, 0)),
            pl.BlockSpec((BQ, D), lambda i: (i + S // BQ, 0)),
        ],
        out_specs=pl.BlockSpec((BQ, D), lambda i: (i, 0)),
        out_shape=jax.ShapeDtypeStruct((S, D), jnp.float32),
    )(xt, g01, g01)

    return out.reshape(1, S, D), router_logits


# speedup vs baseline: 1.4293x; 1.0923x over previous
import functools

import jax
import jax.numpy as jnp
from jax import lax
from jax.experimental import pallas as pl
from jax.experimental.pallas import tpu as pltpu
from jax.experimental.pallas import tpu_sc as plsc

S, D = 2048, 768
H, DK = 12, 64
INNER = H * DK
DFF = 3072
E, TOPK = 8, 2
EPS = 1e-6

BQ = 256
TB = 128
P = 5120
NB = P // TB


def _dot(a, b):
    return jnp.dot(a, b, preferred_element_type=jnp.float32)


_LOG2E = 1.4426950408889634


def _exp(x):
    return jnp.exp2(x * _LOG2E)


def _row_mean_sq(x, d):
    s = x * x
    acc = s[:, 0:128]
    for c in range(128, d, 128):
        acc = acc + s[:, c:c + 128]
    w = 64
    while w >= 1:
        acc = acc[:, :w] + acc[:, w:2 * w]
        w //= 2
    return acc / d


def _ln_qkv_body(x_ref, lnw_ref, w_ref, out_ref):
    x = x_ref[...]
    var = _row_mean_sq(x, D)
    normed = x * lax.rsqrt(var + EPS) * lnw_ref[...]
    out_ref[...] = _dot(normed, w_ref[...])


def _attn_body(q_ref, k_ref, v_ref, o_ref):
    outs = []
    for h in range(2):
        q = q_ref[:, h * DK:(h + 1) * DK]
        k = k_ref[:, h * DK:(h + 1) * DK]
        s = lax.dot_general(q, k, (((1,), (1,)), ((), ())),
                            preferred_element_type=jnp.float32)
        m = jnp.max(s, axis=1, keepdims=True)
        p = _exp(s - m)
        av = _dot(p, v_ref[:, h * DK:(h + 1) * DK])
        outs.append(av / jnp.sum(p, axis=1, keepdims=True))
    o_ref[...] = jnp.concatenate(outs, axis=1)


def _proj_router_body(ctx_ref, x_ref, wo_ref, rw_ref,
                      xt_ref, lg_ref, w2_ref, i2_ref):
    xt = x_ref[...] + _dot(ctx_ref[...], wo_ref[...])
    xt_ref[...] = xt
    lg = _dot(xt, rw_ref[...])
    lg_ref[...] = lg
    mx = jnp.max(lg, axis=1, keepdims=True)
    ex = _exp(lg - mx)
    p = ex / jnp.sum(ex, axis=1, keepdims=True)
    iota = lax.broadcasted_iota(jnp.int32, p.shape, 1)
    m1 = jnp.max(p, axis=1, keepdims=True)
    i1 = jnp.min(jnp.where(p >= m1, iota, E), axis=1, keepdims=True)
    pm = jnp.where(iota == i1, -1.0, p)
    m2 = jnp.max(pm, axis=1, keepdims=True)
    i2 = jnp.min(jnp.where(pm >= m2, iota, E), axis=1, keepdims=True)
    tot = m1 + m2
    w2_ref[...] = jnp.concatenate([m1 / tot, m2 / tot], axis=1)
    i2_ref[...] = jnp.concatenate([i1, i2], axis=1)


def _ffn_body(eid_ref, x_ref, lnw_ref, wi_ref, wo_ref, wr_ref, out_ref):
    x = x_ref[...]
    var = jnp.mean(x * x, axis=1, keepdims=True)
    h = x * lax.rsqrt(var + EPS) * lnw_ref[0]
    a = jnp.maximum(_dot(h, wi_ref[0]), 0.0)
    c = _dot(a, wo_ref[0])
    out_ref[...] = c * wr_ref[:, :1]


def _combine_body(xt_ref, g0_ref, g1_ref, o_ref):
    o_ref[...] = xt_ref[...] + g0_ref[...] + g1_ref[...]


_SC_CORES = 2
_SC_SUBCORES = 16


def _make_sc_gather(n_rows, d, b_idx, chunk):
    nw = _SC_CORES * _SC_SUBCORES
    b_per_w = b_idx // nw
    nch = b_per_w // chunk
    assert b_per_w % chunk == 0 and chunk <= 160 and chunk % 8 == 0
    mesh = plsc.VectorSubcoreMesh(core_axis_name="c", subcore_axis_name="s")

    @functools.partial(
        pl.kernel,
        out_type=jax.ShapeDtypeStruct((b_idx, d), jnp.float32),
        mesh=mesh,
        scratch_types=[
            pltpu.VMEM((chunk,), jnp.int32),
            pltpu.VMEM((chunk, d), jnp.float32),
            pltpu.SemaphoreType.DMA,
        ],
    )
    def gather(table_hbm, idx_hbm, out_hbm, idx_v, rows_v, sem):
        wid = lax.axis_index("s") * _SC_CORES + lax.axis_index("c")
        base = wid * b_per_w
        for c in range(nch):
            off = base + c * chunk
            pltpu.sync_copy(idx_hbm.at[pl.ds(off, chunk)], idx_v)
            pltpu.async_copy(table_hbm.at[idx_v], rows_v, sem).wait()
            pltpu.sync_copy(rows_v, out_hbm.at[pl.ds(off, chunk)])

    return gather


_gather_dispatch = _make_sc_gather(S, D, P, 160)
_gather_return = _make_sc_gather(P, D, 2 * S, 128)


def kernel(hidden_states, ln_attn_w, Wq, Wk, Wv, Wo, ln_ff_w, Wi, Wo_ff,
           router_W):
    x = hidden_states.reshape(S, D)
    wqkv = jnp.concatenate([Wq, Wk, Wv], axis=1)

    qkv = pl.pallas_call(
        _ln_qkv_body,
        grid=(S // BQ,),
        in_specs=[
            pl.BlockSpec((BQ, D), lambda i: (i, 0)),
            pl.BlockSpec((1, D), lambda i: (0, 0)),
            pl.BlockSpec((D, 3 * INNER), lambda i: (0, 0)),
        ],
        out_specs=pl.BlockSpec((BQ, 3 * INNER), lambda i: (i, 0)),
        out_shape=jax.ShapeDtypeStruct((S, 3 * INNER), jnp.float32),
    )(x, ln_attn_w.reshape(1, D), wqkv)

    HP = H // 2
    BA = 512
    ctx = pl.pallas_call(
        _attn_body,
        grid=(HP, S // BA),
        in_specs=[
            pl.BlockSpec((BA, 2 * DK), lambda h, i: (i, h)),
            pl.BlockSpec((S, 2 * DK), lambda h, i: (0, HP + h)),
            pl.BlockSpec((S, 2 * DK), lambda h, i: (0, 2 * HP + h)),
        ],
        out_specs=pl.BlockSpec((BA, 2 * DK), lambda h, i: (i, h)),
        out_shape=jax.ShapeDtypeStruct((S, INNER), jnp.float32),
    )(qkv, qkv, qkv)

    xt, router_logits, w2, i2 = pl.pallas_call(
        _proj_router_body,
        grid=(S // BQ,),
        in_specs=[
            pl.BlockSpec((BQ, INNER), lambda i: (i, 0)),
            pl.BlockSpec((BQ, D), lambda i: (i, 0)),
            pl.BlockSpec((INNER, D), lambda i: (0, 0)),
            pl.BlockSpec((D, E), lambda i: (0, 0)),
        ],
        out_specs=[
            pl.BlockSpec((BQ, D), lambda i: (i, 0)),
            pl.BlockSpec((BQ, E), lambda i: (i, 0)),
            pl.BlockSpec((BQ, TOPK), lambda i: (i, 0)),
            pl.BlockSpec((BQ, TOPK), lambda i: (i, 0)),
        ],
        out_shape=[
            jax.ShapeDtypeStruct((S, D), jnp.float32),
            jax.ShapeDtypeStruct((S, E), jnp.float32),
            jax.ShapeDtypeStruct((S, TOPK), jnp.float32),
            jax.ShapeDtypeStruct((S, TOPK), jnp.int32),
        ],
    )(ctx, x, Wo, router_W)

    e_flat = i2.reshape(-1)
    w_flat = w2.reshape(-1)
    onehot = (e_flat[:, None] == jnp.arange(E)).astype(jnp.int32)
    cnt = onehot.sum(0)
    rank = jnp.cumsum(onehot, axis=0) - onehot
    rank_i = jnp.take_along_axis(rank, e_flat[:, None], axis=1)[:, 0]
    pcnt = ((cnt + TB - 1) // TB) * TB
    poff = jnp.concatenate([jnp.zeros((1,), jnp.int32),
                            jnp.cumsum(pcnt)[:-1].astype(jnp.int32)])
    pos = poff[e_flat] + rank_i
    srctok = jnp.zeros((P,), jnp.int32).at[pos].set(
        jnp.arange(2 * S, dtype=jnp.int32) // 2)
    w_sorted = jnp.zeros((P,), jnp.float32).at[pos].set(w_flat)
    pend = (poff + pcnt).astype(jnp.int32)
    blk_eid = jnp.clip(
        jnp.sum((jnp.arange(NB, dtype=jnp.int32)[:, None] * TB) >= pend[None, :],
                axis=1), 0, E - 1).astype(jnp.int32)
    wrow = jnp.broadcast_to(w_sorted[:, None], (P, 128))
    pos01 = jnp.concatenate([pos[0::2], pos[1::2]])

    sorted_x = _gather_dispatch(xt, srctok)

    contrib = pl.pallas_call(
        _ffn_body,
        grid_spec=pltpu.PrefetchScalarGridSpec(
            num_scalar_prefetch=1,
            grid=(NB,),
            in_specs=[
                pl.BlockSpec((TB, D), lambda b, eid: (b, 0)),
                pl.BlockSpec((1, 1, D), lambda b, eid: (eid[b], 0, 0)),
                pl.BlockSpec((1, D, DFF), lambda b, eid: (eid[b], 0, 0)),
                pl.BlockSpec((1, DFF, D), lambda b, eid: (eid[b], 0, 0)),
                pl.BlockSpec((TB, 128), lambda b, eid: (b, 0)),
            ],
            out_specs=pl.BlockSpec((TB, D), lambda b, eid: (b, 0)),
        ),
        out_shape=jax.ShapeDtypeStruct((P, D), jnp.float32),
    )(blk_eid, sorted_x, ln_ff_w.reshape(E, 1, D), Wi, Wo_ff, wrow)

    g01 = _gather_return(contrib, pos01)

    out = pl.pallas_call(
        _combine_body,
        grid=(S // BQ,),
        in_specs=[
            pl.BlockSpec((BQ, D), lambda i: (i, 0)),
            pl.BlockSpec((BQ, D), lambda i: (i, 0)),
            pl.BlockSpec((BQ, D), lambda i: (i + S // BQ, 0)),
        ],
        out_specs=pl.BlockSpec((BQ, D), lambda i: (i, 0)),
        out_shape=jax.ShapeDtypeStruct((S, D), jnp.float32),
    )(xt, g01, g01)

    return out.reshape(1, S, D), router_logits


# speedup vs baseline: 1.4536x; 1.0170x over previous
import functools

import jax
import jax.numpy as jnp
from jax import lax
from jax.experimental import pallas as pl
from jax.experimental.pallas import tpu as pltpu
from jax.experimental.pallas import tpu_sc as plsc

S, D = 2048, 768
H, DK = 12, 64
INNER = H * DK
DFF = 3072
E, TOPK = 8, 2
EPS = 1e-6

BQ = 512
TB = 128
P = 5120
NB = P // TB


def _dot(a, b):
    return jnp.dot(a, b, preferred_element_type=jnp.float32)


_LOG2E = 1.4426950408889634


def _exp(x):
    return jnp.exp2(x * _LOG2E)


def _row_mean_sq(x, d):
    s = x * x
    acc = s[:, 0:128]
    for c in range(128, d, 128):
        acc = acc + s[:, c:c + 128]
    w = 64
    while w >= 1:
        acc = acc[:, :w] + acc[:, w:2 * w]
        w //= 2
    return acc / d


def _ln_qkv_body(x_ref, lnw_ref, w_ref, out_ref):
    x = x_ref[...]
    var = _row_mean_sq(x, D)
    normed = x * lax.rsqrt(var + EPS) * lnw_ref[...]
    out_ref[...] = _dot(normed, w_ref[...])


def _attn_body(q_ref, k_ref, v_ref, o_ref):
    outs = []
    for h in range(2):
        q = q_ref[:, h * DK:(h + 1) * DK]
        k = k_ref[:, h * DK:(h + 1) * DK]
        s = lax.dot_general(q, k, (((1,), (1,)), ((), ())),
                            preferred_element_type=jnp.float32)
        m = jnp.max(s, axis=1, keepdims=True)
        p = _exp(s - m)
        av = _dot(p, v_ref[:, h * DK:(h + 1) * DK])
        outs.append(av / jnp.sum(p, axis=1, keepdims=True))
    o_ref[...] = jnp.concatenate(outs, axis=1)


def _proj_router_body(ctx_ref, x_ref, wo_ref, rw_ref,
                      xt_ref, lg_ref, w2_ref, i2_ref):
    xt = x_ref[...] + _dot(ctx_ref[...], wo_ref[...])
    xt_ref[...] = xt
    lg = _dot(xt, rw_ref[...])
    lg_ref[...] = lg
    mx = jnp.max(lg, axis=1, keepdims=True)
    ex = _exp(lg - mx)
    p = ex / jnp.sum(ex, axis=1, keepdims=True)
    iota = lax.broadcasted_iota(jnp.int32, p.shape, 1)
    m1 = jnp.max(p, axis=1, keepdims=True)
    i1 = jnp.min(jnp.where(p >= m1, iota, E), axis=1, keepdims=True)
    pm = jnp.where(iota == i1, -1.0, p)
    m2 = jnp.max(pm, axis=1, keepdims=True)
    i2 = jnp.min(jnp.where(pm >= m2, iota, E), axis=1, keepdims=True)
    tot = m1 + m2
    w2_ref[...] = jnp.concatenate([m1 / tot, m2 / tot], axis=1)
    i2_ref[...] = jnp.concatenate([i1, i2], axis=1)


def _ffn_body(eid_ref, x_ref, lnw_ref, wi_ref, wo_ref, wr_ref, out_ref):
    x = x_ref[...]
    var = jnp.mean(x * x, axis=1, keepdims=True)
    h = x * lax.rsqrt(var + EPS) * lnw_ref[0]
    a = jnp.maximum(_dot(h, wi_ref[0]), 0.0)
    c = _dot(a, wo_ref[0])
    out_ref[...] = c * wr_ref[:, :1]


def _combine_body(xt_ref, g0_ref, g1_ref, o_ref):
    o_ref[...] = xt_ref[...] + g0_ref[...] + g1_ref[...]


_SC_CORES = 2
_SC_SUBCORES = 16


def _make_sc_gather(n_rows, d, b_idx, chunk):
    nw = _SC_CORES * _SC_SUBCORES
    b_per_w = b_idx // nw
    nch = b_per_w // chunk
    assert b_per_w % chunk == 0 and chunk <= 160 and chunk % 8 == 0
    mesh = plsc.VectorSubcoreMesh(core_axis_name="c", subcore_axis_name="s")

    @functools.partial(
        pl.kernel,
        out_type=jax.ShapeDtypeStruct((b_idx, d), jnp.float32),
        mesh=mesh,
        scratch_types=[
            pltpu.VMEM((chunk,), jnp.int32),
            pltpu.VMEM((chunk, d), jnp.float32),
            pltpu.SemaphoreType.DMA,
        ],
    )
    def gather(table_hbm, idx_hbm, out_hbm, idx_v, rows_v, sem):
        wid = lax.axis_index("s") * _SC_CORES + lax.axis_index("c")
        base = wid * b_per_w
        for c in range(nch):
            off = base + c * chunk
            pltpu.sync_copy(idx_hbm.at[pl.ds(off, chunk)], idx_v)
            pltpu.async_copy(table_hbm.at[idx_v], rows_v, sem).wait()
            pltpu.sync_copy(rows_v, out_hbm.at[pl.ds(off, chunk)])

    return gather


_gather_dispatch = _make_sc_gather(S, D, P, 160)
_gather_return = _make_sc_gather(P, D, 2 * S, 128)


def kernel(hidden_states, ln_attn_w, Wq, Wk, Wv, Wo, ln_ff_w, Wi, Wo_ff,
           router_W):
    x = hidden_states.reshape(S, D)
    wqkv = jnp.concatenate([Wq, Wk, Wv], axis=1)

    qkv = pl.pallas_call(
        _ln_qkv_body,
        grid=(S // BQ,),
        in_specs=[
            pl.BlockSpec((BQ, D), lambda i: (i, 0)),
            pl.BlockSpec((1, D), lambda i: (0, 0)),
            pl.BlockSpec((D, 3 * INNER), lambda i: (0, 0)),
        ],
        out_specs=pl.BlockSpec((BQ, 3 * INNER), lambda i: (i, 0)),
        out_shape=jax.ShapeDtypeStruct((S, 3 * INNER), jnp.float32),
    )(x, ln_attn_w.reshape(1, D), wqkv)

    HP = H // 2
    BA = 512
    ctx = pl.pallas_call(
        _attn_body,
        grid=(HP, S // BA),
        in_specs=[
            pl.BlockSpec((BA, 2 * DK), lambda h, i: (i, h)),
            pl.BlockSpec((S, 2 * DK), lambda h, i: (0, HP + h)),
            pl.BlockSpec((S, 2 * DK), lambda h, i: (0, 2 * HP + h)),
        ],
        out_specs=pl.BlockSpec((BA, 2 * DK), lambda h, i: (i, h)),
        out_shape=jax.ShapeDtypeStruct((S, INNER), jnp.float32),
    )(qkv, qkv, qkv)

    xt, router_logits, w2, i2 = pl.pallas_call(
        _proj_router_body,
        grid=(S // BQ,),
        in_specs=[
            pl.BlockSpec((BQ, INNER), lambda i: (i, 0)),
            pl.BlockSpec((BQ, D), lambda i: (i, 0)),
            pl.BlockSpec((INNER, D), lambda i: (0, 0)),
            pl.BlockSpec((D, E), lambda i: (0, 0)),
        ],
        out_specs=[
            pl.BlockSpec((BQ, D), lambda i: (i, 0)),
            pl.BlockSpec((BQ, E), lambda i: (i, 0)),
            pl.BlockSpec((BQ, TOPK), lambda i: (i, 0)),
            pl.BlockSpec((BQ, TOPK), lambda i: (i, 0)),
        ],
        out_shape=[
            jax.ShapeDtypeStruct((S, D), jnp.float32),
            jax.ShapeDtypeStruct((S, E), jnp.float32),
            jax.ShapeDtypeStruct((S, TOPK), jnp.float32),
            jax.ShapeDtypeStruct((S, TOPK), jnp.int32),
        ],
    )(ctx, x, Wo, router_W)

    e_flat = i2.reshape(-1)
    w_flat = w2.reshape(-1)
    onehot = (e_flat[:, None] == jnp.arange(E)).astype(jnp.int32)
    cnt = onehot.sum(0)
    rank = jnp.cumsum(onehot, axis=0) - onehot
    rank_i = jnp.take_along_axis(rank, e_flat[:, None], axis=1)[:, 0]
    pcnt = ((cnt + TB - 1) // TB) * TB
    poff = jnp.concatenate([jnp.zeros((1,), jnp.int32),
                            jnp.cumsum(pcnt)[:-1].astype(jnp.int32)])
    pos = poff[e_flat] + rank_i
    srctok = jnp.zeros((P,), jnp.int32).at[pos].set(
        jnp.arange(2 * S, dtype=jnp.int32) // 2)
    w_sorted = jnp.zeros((P,), jnp.float32).at[pos].set(w_flat)
    pend = (poff + pcnt).astype(jnp.int32)
    blk_eid = jnp.clip(
        jnp.sum((jnp.arange(NB, dtype=jnp.int32)[:, None] * TB) >= pend[None, :],
                axis=1), 0, E - 1).astype(jnp.int32)
    wrow = jnp.broadcast_to(w_sorted[:, None], (P, 128))
    pos01 = jnp.concatenate([pos[0::2], pos[1::2]])

    sorted_x = _gather_dispatch(xt, srctok)

    contrib = pl.pallas_call(
        _ffn_body,
        grid_spec=pltpu.PrefetchScalarGridSpec(
            num_scalar_prefetch=1,
            grid=(NB,),
            in_specs=[
                pl.BlockSpec((TB, D), lambda b, eid: (b, 0)),
                pl.BlockSpec((1, 1, D), lambda b, eid: (eid[b], 0, 0)),
                pl.BlockSpec((1, D, DFF), lambda b, eid: (eid[b], 0, 0)),
                pl.BlockSpec((1, DFF, D), lambda b, eid: (eid[b], 0, 0)),
                pl.BlockSpec((TB, 128), lambda b, eid: (b, 0)),
            ],
            out_specs=pl.BlockSpec((TB, D), lambda b, eid: (b, 0)),
        ),
        out_shape=jax.ShapeDtypeStruct((P, D), jnp.float32),
    )(blk_eid, sorted_x, ln_ff_w.reshape(E, 1, D), Wi, Wo_ff, wrow)

    g01 = _gather_return(contrib, pos01)

    out = pl.pallas_call(
        _combine_body,
        grid=(S // BQ,),
        in_specs=[
            pl.BlockSpec((BQ, D), lambda i: (i, 0)),
            pl.BlockSpec((BQ, D), lambda i: (i, 0)),
            pl.BlockSpec((BQ, D), lambda i: (i + S // BQ, 0)),
        ],
        out_specs=pl.BlockSpec((BQ, D), lambda i: (i, 0)),
        out_shape=jax.ShapeDtypeStruct((S, D), jnp.float32),
    )(xt, g01, g01)

    return out.reshape(1, S, D), router_logits
